# baseline (speedup 1.0000x reference)
import functools

import jax
import jax.numpy as jnp
from jax import lax
from jax.experimental import pallas as pl
from jax.experimental.pallas import tpu as pltpu
from jax.experimental.pallas import tpu_sc as plsc

N_NODES = 10000
D_FEAT = 128
N_EDGES = 320000

NUM_CORES = 2
NUM_SUBCORES = 16
NUM_WORKERS = NUM_CORES * NUM_SUBCORES

CHUNK = 80
SEG = 8
NSEG = 16
CHUNKS_PER_TILE = SEG * NSEG
EDGES_PER_TILE = CHUNK * CHUNKS_PER_TILE
PAD_EDGES = NUM_WORKERS * EDGES_PER_TILE - N_EDGES

ACC_ROWS = 10080
ZCHUNK = 80
N_ZCHUNKS = ACC_ROWS // ZCHUNK
WCHUNK = 80
N_WCHUNKS = N_NODES // WCHUNK


def _sc_kernel_body(src_hbm, dst_hbm, x_hbm, part_hbm,
                    acc, srcv, dstv, buf0, buf1, gsem0, gsem1, isem):
    c = lax.axis_index("c")
    s = lax.axis_index("s")
    wid = c * NUM_SUBCORES + s

    import contextlib
    def zero_row(r, _):
        for k in range(D_FEAT // 16):
            buf0[r, pl.ds(k * 16, 16)] = jnp.zeros((16,), jnp.float32)
        return _
    with jax.named_scope("ph_zbuf"):
        lax.fori_loop(0, ZCHUNK, zero_row, None)

    zctx = jax.named_scope("ph_zacc"); zctx.__enter__()
    for k in range((N_ZCHUNKS + NUM_SUBCORES - 1) // NUM_SUBCORES):
        j = s + k * NUM_SUBCORES

        @pl.when(j < N_ZCHUNKS)
        def _():
            pltpu.sync_copy(buf0, acc.at[pl.ds(j * ZCHUNK, ZCHUNK)])
    zctx.__exit__(None, None, None)

    with jax.named_scope("ph_idx0"):
        pltpu.sync_copy(src_hbm.at[wid, pl.ds(0, SEG)], srcv.at[pl.ds(0, SEG)])
        pltpu.sync_copy(dst_hbm.at[wid, pl.ds(0, SEG)], dstv.at[pl.ds(0, SEG)])

    with jax.named_scope("ph_barrier1"):
        plsc.subcore_barrier()

    def start_gather(idx_row, buf, sem):
        pltpu.async_copy(x_hbm.at[idx_row], buf, sem)

    def wait_gather(idx_row, buf, sem):
        pltpu.make_async_copy(x_hbm.at[idx_row], buf, sem).wait()

    def scatter(idx_row, buf):
        pltpu.sync_copy(buf, acc.at[idx_row], add=True)

    start_gather(srcv.at[0], buf0, gsem0)
    start_gather(srcv.at[1], buf1, gsem1)

    def seg_body(sg, _):
        b = lax.rem(sg, 2) * SEG
        nb = lax.rem(sg + 1, 2) * SEG

        @pl.when(sg + 1 < NSEG)
        def _():
            pltpu.async_copy(src_hbm.at[wid, pl.ds((sg + 1) * SEG, SEG)],
                             srcv.at[pl.ds(nb, SEG)], isem)
            pltpu.async_copy(dst_hbm.at[wid, pl.ds((sg + 1) * SEG, SEG)],
                             dstv.at[pl.ds(nb, SEG)], isem)

        for k in range(SEG):
            buf, sem = (buf0, gsem0) if k % 2 == 0 else (buf1, gsem1)
            wait_gather(srcv.at[b + k], buf, sem)
            scatter(dstv.at[b + k], buf)
            if k < SEG - 2:
                start_gather(srcv.at[b + k + 2], buf, sem)
            else:
                @pl.when(sg + 1 < NSEG)
                def _():
                    if k == SEG - 2:
                        pltpu.make_async_copy(
                            src_hbm.at[wid, pl.ds((sg + 1) * SEG, SEG)],
                            srcv.at[pl.ds(nb, SEG)], isem).wait()
                        pltpu.make_async_copy(
                            dst_hbm.at[wid, pl.ds((sg + 1) * SEG, SEG)],
                            dstv.at[pl.ds(nb, SEG)], isem).wait()
                    start_gather(srcv.at[nb + k - (SEG - 2)], buf, sem)
        return _
    with jax.named_scope("ph_main"):
        lax.fori_loop(0, NSEG, seg_body, None)

    with jax.named_scope("ph_barrier2"):
        plsc.subcore_barrier()

    ctx = jax.named_scope("ph_writeout"); ctx.__enter__()
    for k in range((N_WCHUNKS + NUM_SUBCORES - 1) // NUM_SUBCORES):
        j = s + k * NUM_SUBCORES

        @pl.when(j < N_WCHUNKS)
        def _():
            pltpu.sync_copy(acc.at[pl.ds(j * WCHUNK, WCHUNK)],
                            buf0.at[pl.ds(0, WCHUNK)])
            pltpu.sync_copy(buf0.at[pl.ds(0, WCHUNK)],
                            part_hbm.at[c, pl.ds(j * WCHUNK, WCHUNK)])
    ctx.__exit__(None, None, None)


_sc_scatter_gather = functools.partial(
    pl.kernel,
    out_type=jax.ShapeDtypeStruct((NUM_CORES, N_NODES, D_FEAT), jnp.float32),
    mesh=plsc.VectorSubcoreMesh(core_axis_name="c", subcore_axis_name="s"),
    scratch_types=[
        pltpu.VMEM_SHARED((ACC_ROWS, D_FEAT), jnp.float32),
        pltpu.VMEM((2 * SEG, CHUNK), jnp.int32),
        pltpu.VMEM((2 * SEG, CHUNK), jnp.int32),
        pltpu.VMEM((CHUNK, D_FEAT), jnp.float32),
        pltpu.VMEM((CHUNK, D_FEAT), jnp.float32),
        pltpu.SemaphoreType.DMA,
        pltpu.SemaphoreType.DMA,
        pltpu.SemaphoreType.DMA,
    ],
)(_sc_kernel_body)


def _add_body(a_ref, b_ref, o_ref):
    o_ref[...] = a_ref[0] + b_ref[0]


def _combine(partials):
    rows_per_blk = N_NODES // 10
    return pl.pallas_call(
        _add_body,
        out_shape=jax.ShapeDtypeStruct((N_NODES, D_FEAT), jnp.float32),
        grid=(10,),
        in_specs=[
            pl.BlockSpec((1, rows_per_blk, D_FEAT), lambda i: (0, i, 0)),
            pl.BlockSpec((1, rows_per_blk, D_FEAT), lambda i: (1, i, 0)),
        ],
        out_specs=pl.BlockSpec((rows_per_blk, D_FEAT), lambda i: (i, 0)),
    )(partials, partials)


def kernel(x, edge_index):
    src = edge_index[0].astype(jnp.int32)
    dst = edge_index[1].astype(jnp.int32)
    pad_dst = N_NODES + jnp.arange(PAD_EDGES, dtype=jnp.int32) % (ACC_ROWS - N_NODES)
    src = jnp.concatenate([src, jnp.zeros((PAD_EDGES,), jnp.int32)])
    dst = jnp.concatenate([dst, pad_dst])
    src = src.reshape(NUM_WORKERS, CHUNKS_PER_TILE, CHUNK)
    dst = dst.reshape(NUM_WORKERS, CHUNKS_PER_TILE, CHUNK)
    partials = _sc_scatter_gather(src, dst, x)
    return _combine(partials)


# speedup vs baseline: 3.2512x; 3.2512x over previous
import functools

import jax
import jax.numpy as jnp
from jax import lax
from jax.experimental import pallas as pl
from jax.experimental.pallas import tpu as pltpu
from jax.experimental.pallas import tpu_sc as plsc

N_NODES = 10000
D_FEAT = 128
N_EDGES = 320000

NUM_CORES = 2
NUM_SUBCORES = 16
NUM_WORKERS = NUM_CORES * NUM_SUBCORES

CHUNK = 80
SEG = 8
NSEG = 16
CHUNKS_PER_TILE = SEG * NSEG
EDGES_PER_TILE = CHUNK * CHUNKS_PER_TILE
PAD_EDGES = NUM_WORKERS * EDGES_PER_TILE - N_EDGES

ACC_ROWS = 10080
ZCHUNK = 80
N_ZCHUNKS = ACC_ROWS // ZCHUNK
WCHUNK = 80
N_WCHUNKS = N_NODES // WCHUNK


def _sc_kernel_body(src_hbm, dst_hbm, x_hbm, part_hbm,
                    acc, srcv, dstv, buf0, buf1, gsem0, gsem1, isem):
    c = lax.axis_index("c")
    s = lax.axis_index("s")
    wid = c * NUM_SUBCORES + s

    import contextlib
    def zero_row(r, _):
        for k in range(D_FEAT // 16):
            buf0[r, pl.ds(k * 16, 16)] = jnp.zeros((16,), jnp.float32)
        return _
    with jax.named_scope("ph_zbuf"):
        lax.fori_loop(0, ZCHUNK, zero_row, None)

    zctx = jax.named_scope("ph_zacc"); zctx.__enter__()
    for k in range((N_ZCHUNKS + NUM_SUBCORES - 1) // NUM_SUBCORES):
        j = s + k * NUM_SUBCORES

        @pl.when(j < N_ZCHUNKS)
        def _():
            pltpu.sync_copy(buf0, acc.at[pl.ds(j * ZCHUNK, ZCHUNK)])
    zctx.__exit__(None, None, None)

    with jax.named_scope("ph_idx0"):
        pltpu.sync_copy(src_hbm.at[wid, pl.ds(0, SEG)], srcv.at[pl.ds(0, SEG)])
        pltpu.sync_copy(dst_hbm.at[wid, pl.ds(0, SEG)], dstv.at[pl.ds(0, SEG)])

    with jax.named_scope("ph_barrier1"):
        plsc.subcore_barrier()

    def start_gather(idx_row, buf, sem):
        pltpu.async_copy(x_hbm.at[idx_row], buf, sem)

    def wait_gather(idx_row, buf, sem):
        pltpu.make_async_copy(x_hbm.at[idx_row], buf, sem).wait()

    def scatter(idx_row, buf):
        pltpu.sync_copy(buf, acc.at[idx_row], add=True)

    start_gather(srcv.at[0], buf0, gsem0)
    start_gather(srcv.at[1], buf1, gsem1)

    def seg_body(sg, _):
        b = lax.rem(sg, 2) * SEG
        nb = lax.rem(sg + 1, 2) * SEG

        @pl.when(sg + 1 < NSEG)
        def _():
            pltpu.async_copy(src_hbm.at[wid, pl.ds((sg + 1) * SEG, SEG)],
                             srcv.at[pl.ds(nb, SEG)], isem)
            pltpu.async_copy(dst_hbm.at[wid, pl.ds((sg + 1) * SEG, SEG)],
                             dstv.at[pl.ds(nb, SEG)], isem)

        for k in range(SEG):
            buf, sem = (buf0, gsem0) if k % 2 == 0 else (buf1, gsem1)
            wait_gather(srcv.at[b + k], buf, sem)
            scatter(dstv.at[b + k], buf)
            if k < SEG - 2:
                start_gather(srcv.at[b + k + 2], buf, sem)
            else:
                @pl.when(sg + 1 < NSEG)
                def _():
                    if k == SEG - 2:
                        pltpu.make_async_copy(
                            src_hbm.at[wid, pl.ds((sg + 1) * SEG, SEG)],
                            srcv.at[pl.ds(nb, SEG)], isem).wait()
                        pltpu.make_async_copy(
                            dst_hbm.at[wid, pl.ds((sg + 1) * SEG, SEG)],
                            dstv.at[pl.ds(nb, SEG)], isem).wait()
                    start_gather(srcv.at[nb + k - (SEG - 2)], buf, sem)
        return _
    with jax.named_scope("ph_main"):
        lax.fori_loop(0, NSEG, seg_body, None)

    with jax.named_scope("ph_barrier2"):
        plsc.subcore_barrier()

    ctx = jax.named_scope("ph_writeout"); ctx.__enter__()
    for k in range((N_WCHUNKS + NUM_SUBCORES - 1) // NUM_SUBCORES):
        j = s + k * NUM_SUBCORES

        @pl.when(j < N_WCHUNKS)
        def _():
            pltpu.sync_copy(acc.at[pl.ds(j * WCHUNK, WCHUNK)],
                            buf0.at[pl.ds(0, WCHUNK)])
            pltpu.sync_copy(buf0.at[pl.ds(0, WCHUNK)],
                            part_hbm.at[c, pl.ds(j * WCHUNK, WCHUNK)])
    ctx.__exit__(None, None, None)


_sc_scatter_gather = functools.partial(
    pl.kernel,
    out_type=jax.ShapeDtypeStruct((NUM_CORES, N_NODES, D_FEAT), jnp.float32),
    mesh=plsc.VectorSubcoreMesh(core_axis_name="c", subcore_axis_name="s"),
    scratch_types=[
        pltpu.VMEM_SHARED((ACC_ROWS, D_FEAT), jnp.float32),
        pltpu.VMEM((2 * SEG, CHUNK), jnp.int32),
        pltpu.VMEM((2 * SEG, CHUNK), jnp.int32),
        pltpu.VMEM((CHUNK, D_FEAT), jnp.float32),
        pltpu.VMEM((CHUNK, D_FEAT), jnp.float32),
        pltpu.SemaphoreType.DMA,
        pltpu.SemaphoreType.DMA,
        pltpu.SemaphoreType.DMA,
    ],
)(_sc_kernel_body)


def _add_body(a_ref, b_ref, o_ref):
    o_ref[...] = a_ref[0] + b_ref[0]


def _combine(partials):
    rows_per_blk = N_NODES // 10
    return pl.pallas_call(
        _add_body,
        out_shape=jax.ShapeDtypeStruct((N_NODES, D_FEAT), jnp.float32),
        grid=(10,),
        in_specs=[
            pl.BlockSpec((1, rows_per_blk, D_FEAT), lambda i: (0, i, 0)),
            pl.BlockSpec((1, rows_per_blk, D_FEAT), lambda i: (1, i, 0)),
        ],
        out_specs=pl.BlockSpec((rows_per_blk, D_FEAT), lambda i: (i, 0)),
    )(partials, partials)


def kernel(x, edge_index):
    src = edge_index[0].astype(jnp.int32)
    dst = edge_index[1].astype(jnp.int32)
    pad_per_tile = PAD_EDGES // NUM_WORKERS
    real_per_tile = N_EDGES // NUM_WORKERS
    lane = jnp.arange(pad_per_tile, dtype=jnp.int32)
    pad_src = jnp.broadcast_to(lane % N_NODES, (NUM_WORKERS, pad_per_tile))
    pad_dst = jnp.broadcast_to(
        N_NODES + lane % (ACC_ROWS - N_NODES), (NUM_WORKERS, pad_per_tile))
    src = jnp.concatenate(
        [src.reshape(NUM_WORKERS, real_per_tile), pad_src], axis=1)
    dst = jnp.concatenate(
        [dst.reshape(NUM_WORKERS, real_per_tile), pad_dst], axis=1)
    src = src.reshape(NUM_WORKERS, CHUNKS_PER_TILE, CHUNK)
    dst = dst.reshape(NUM_WORKERS, CHUNKS_PER_TILE, CHUNK)
    partials = _sc_scatter_gather(src, dst, x)
    return _combine(partials)


# speedup vs baseline: 3.3758x; 1.0383x over previous
import functools

import jax
import jax.numpy as jnp
from jax import lax
from jax.experimental import pallas as pl
from jax.experimental.pallas import tpu as pltpu
from jax.experimental.pallas import tpu_sc as plsc

N_NODES = 10000
D_FEAT = 128
N_EDGES = 320000

NUM_CORES = 2
NUM_SUBCORES = 16
NUM_WORKERS = NUM_CORES * NUM_SUBCORES

CHUNK = 80
SEG = 8
NSEG = 16
CHUNKS_PER_TILE = SEG * NSEG
EDGES_PER_TILE = CHUNK * CHUNKS_PER_TILE
PAD_EDGES = NUM_WORKERS * EDGES_PER_TILE - N_EDGES

ACC_ROWS = 10080
ZCHUNK = 80
N_ZCHUNKS = ACC_ROWS // ZCHUNK
WCHUNK = 80
N_WCHUNKS = N_NODES // WCHUNK


def _sc_kernel_body(src_hbm, dst_hbm, x_hbm, part_hbm,
                    acc, srcv, dstv, b0, b1, b2, b3,
                    g0, g1, g2, g3, s0, s1, s2, s3, isem, w0, w1):
    c = lax.axis_index("c")
    s = lax.axis_index("s")
    wid = c * NUM_SUBCORES + s
    bufs = (b0, b1, b2, b3)
    gsems = (g0, g1, g2, g3)
    ssems = (s0, s1, s2, s3)
    wsems = (w0, w1)

    def zero_row(r, _):
        for k in range(D_FEAT // 16):
            b0[r, pl.ds(k * 16, 16)] = jnp.zeros((16,), jnp.float32)
        return _
    lax.fori_loop(0, ZCHUNK, zero_row, None)

    for k in range((N_ZCHUNKS + NUM_SUBCORES - 1) // NUM_SUBCORES):
        j = s + k * NUM_SUBCORES

        @pl.when(j < N_ZCHUNKS)
        def _():
            pltpu.sync_copy(b0, acc.at[pl.ds(j * ZCHUNK, ZCHUNK)])

    pltpu.sync_copy(src_hbm.at[wid, pl.ds(0, SEG)], srcv.at[pl.ds(0, SEG)])
    pltpu.sync_copy(dst_hbm.at[wid, pl.ds(0, SEG)], dstv.at[pl.ds(0, SEG)])

    plsc.subcore_barrier()

    def start_gather(idx_row, buf, sem):
        pltpu.async_copy(x_hbm.at[idx_row], buf, sem)

    def wait_gather(idx_row, buf, sem):
        pltpu.make_async_copy(x_hbm.at[idx_row], buf, sem).wait()

    def start_scatter(idx_row, buf, sem):
        pltpu.async_copy(buf, acc.at[idx_row], sem, add=True)

    def wait_scatter(idx_row, buf, sem):
        pltpu.make_async_copy(buf, acc.at[idx_row], sem).wait()

    start_gather(srcv.at[0], b0, g0)
    start_gather(srcv.at[1], b1, g1)

    def seg_body(sg, _):
        b = lax.rem(sg, 2) * SEG
        nb = lax.rem(sg + 1, 2) * SEG

        @pl.when(sg + 1 < NSEG)
        def _():
            pltpu.async_copy(src_hbm.at[wid, pl.ds((sg + 1) * SEG, SEG)],
                             srcv.at[pl.ds(nb, SEG)], isem)
            pltpu.async_copy(dst_hbm.at[wid, pl.ds((sg + 1) * SEG, SEG)],
                             dstv.at[pl.ds(nb, SEG)], isem)

        for k in range(SEG):
            m = k % 4
            wait_gather(srcv.at[b + k], bufs[m], gsems[m])
            start_scatter(dstv.at[b + k], bufs[m], ssems[m])
            m2 = (k + 2) % 4
            if k >= 2:
                wait_scatter(dstv.at[b + k - 2], bufs[m2], ssems[m2])
            else:
                @pl.when(sg >= 1)
                def _():
                    wait_scatter(dstv.at[b + k], bufs[m2], ssems[m2])
            if k < SEG - 2:
                start_gather(srcv.at[b + k + 2], bufs[m2], gsems[m2])
            else:
                @pl.when(sg + 1 < NSEG)
                def _():
                    if k == SEG - 2:
                        pltpu.make_async_copy(
                            src_hbm.at[wid, pl.ds((sg + 1) * SEG, SEG)],
                            srcv.at[pl.ds(nb, SEG)], isem).wait()
                        pltpu.make_async_copy(
                            dst_hbm.at[wid, pl.ds((sg + 1) * SEG, SEG)],
                            dstv.at[pl.ds(nb, SEG)], isem).wait()
                    start_gather(srcv.at[nb + k - (SEG - 2)], bufs[m2],
                                 gsems[m2])
        return _
    lax.fori_loop(0, NSEG, seg_body, None)

    wait_scatter(dstv.at[SEG + 6], b2, s2)
    wait_scatter(dstv.at[SEG + 7], b3, s3)

    plsc.subcore_barrier()

    for k in range((N_WCHUNKS + NUM_SUBCORES - 1) // NUM_SUBCORES):
        j = s + k * NUM_SUBCORES
        buf, sem = (b0, w0) if k % 2 == 0 else (b1, w1)

        @pl.when(j < N_WCHUNKS)
        def _():
            if k >= 2:
                pltpu.make_async_copy(
                    buf.at[pl.ds(0, WCHUNK)],
                    part_hbm.at[c, pl.ds(j * WCHUNK, WCHUNK)], sem).wait()
            pltpu.sync_copy(acc.at[pl.ds(j * WCHUNK, WCHUNK)],
                            buf.at[pl.ds(0, WCHUNK)])
            pltpu.async_copy(buf.at[pl.ds(0, WCHUNK)],
                             part_hbm.at[c, pl.ds(j * WCHUNK, WCHUNK)], sem)

    for sem, buf in ((w0, b0), (w1, b1)):
        pltpu.make_async_copy(
            buf.at[pl.ds(0, WCHUNK)],
            part_hbm.at[c, pl.ds(0, WCHUNK)], sem).wait()


_sc_scatter_gather = functools.partial(
    pl.kernel,
    out_type=jax.ShapeDtypeStruct((NUM_CORES, N_NODES, D_FEAT), jnp.float32),
    mesh=plsc.VectorSubcoreMesh(core_axis_name="c", subcore_axis_name="s"),
    scratch_types=[
        pltpu.VMEM_SHARED((ACC_ROWS, D_FEAT), jnp.float32),
        pltpu.VMEM((2 * SEG, CHUNK), jnp.int32),
        pltpu.VMEM((2 * SEG, CHUNK), jnp.int32),
        pltpu.VMEM((CHUNK, D_FEAT), jnp.float32),
        pltpu.VMEM((CHUNK, D_FEAT), jnp.float32),
        pltpu.VMEM((CHUNK, D_FEAT), jnp.float32),
        pltpu.VMEM((CHUNK, D_FEAT), jnp.float32),
    ] + [pltpu.SemaphoreType.DMA] * 11,
)(_sc_kernel_body)


def _add_body(a_ref, b_ref, o_ref):
    o_ref[...] = a_ref[0] + b_ref[0]


def _combine(partials):
    rows_per_blk = N_NODES // 10
    return pl.pallas_call(
        _add_body,
        out_shape=jax.ShapeDtypeStruct((N_NODES, D_FEAT), jnp.float32),
        grid=(10,),
        in_specs=[
            pl.BlockSpec((1, rows_per_blk, D_FEAT), lambda i: (0, i, 0)),
            pl.BlockSpec((1, rows_per_blk, D_FEAT), lambda i: (1, i, 0)),
        ],
        out_specs=pl.BlockSpec((rows_per_blk, D_FEAT), lambda i: (i, 0)),
    )(partials, partials)


def kernel(x, edge_index):
    src = edge_index[0].astype(jnp.int32)
    dst = edge_index[1].astype(jnp.int32)
    pad_per_tile = PAD_EDGES // NUM_WORKERS
    real_per_tile = N_EDGES // NUM_WORKERS
    lane = jnp.arange(pad_per_tile, dtype=jnp.int32)
    pad_src = jnp.broadcast_to(lane % N_NODES, (NUM_WORKERS, pad_per_tile))
    pad_dst = jnp.broadcast_to(
        N_NODES + lane % (ACC_ROWS - N_NODES), (NUM_WORKERS, pad_per_tile))
    src = jnp.concatenate(
        [src.reshape(NUM_WORKERS, real_per_tile), pad_src], axis=1)
    dst = jnp.concatenate(
        [dst.reshape(NUM_WORKERS, real_per_tile), pad_dst], axis=1)
    src = src.reshape(NUM_WORKERS, CHUNKS_PER_TILE, CHUNK)
    dst = dst.reshape(NUM_WORKERS, CHUNKS_PER_TILE, CHUNK)
    partials = _sc_scatter_gather(src, dst, x)
    return _combine(partials)


# speedup vs baseline: 3.4092x; 1.0099x over previous
import functools

import jax
import jax.numpy as jnp
from jax import lax
from jax.experimental import pallas as pl
from jax.experimental.pallas import tpu as pltpu
from jax.experimental.pallas import tpu_sc as plsc

N_NODES = 10000
D_FEAT = 128
N_EDGES = 320000

NUM_CORES = 2
NUM_SUBCORES = 16
NUM_WORKERS = NUM_CORES * NUM_SUBCORES

CHUNK = 88
SEG = 8
NSEG = 15
CHUNKS_PER_TILE = SEG * NSEG
EDGES_PER_TILE = CHUNK * CHUNKS_PER_TILE
PAD_EDGES = NUM_WORKERS * EDGES_PER_TILE - N_EDGES

ACC_ROWS = 10120
ZCHUNK = 88
N_ZCHUNKS = ACC_ROWS // ZCHUNK
WCHUNK = 80
N_WCHUNKS = N_NODES // WCHUNK


def _sc_kernel_body(src_hbm, dst_hbm, x_hbm, part_hbm,
                    acc, srcv, dstv, b0, b1, b2, b3,
                    g0, g1, g2, g3, s0, s1, s2, s3, isem, w0, w1):
    c = lax.axis_index("c")
    s = lax.axis_index("s")
    wid = c * NUM_SUBCORES + s
    bufs = (b0, b1, b2, b3)
    gsems = (g0, g1, g2, g3)
    ssems = (s0, s1, s2, s3)
    wsems = (w0, w1)

    def zero_row(r, _):
        for k in range(D_FEAT // 16):
            b0[r, pl.ds(k * 16, 16)] = jnp.zeros((16,), jnp.float32)
        return _
    lax.fori_loop(0, ZCHUNK, zero_row, None)

    for k in range((N_ZCHUNKS + NUM_SUBCORES - 1) // NUM_SUBCORES):
        j = s + k * NUM_SUBCORES

        @pl.when(j < N_ZCHUNKS)
        def _():
            pltpu.sync_copy(b0, acc.at[pl.ds(j * ZCHUNK, ZCHUNK)])

    pltpu.sync_copy(src_hbm.at[wid, pl.ds(0, SEG)], srcv.at[pl.ds(0, SEG)])
    pltpu.sync_copy(dst_hbm.at[wid, pl.ds(0, SEG)], dstv.at[pl.ds(0, SEG)])

    plsc.subcore_barrier()

    def start_gather(idx_row, buf, sem):
        pltpu.async_copy(x_hbm.at[idx_row], buf, sem)

    def wait_gather(idx_row, buf, sem):
        pltpu.make_async_copy(x_hbm.at[idx_row], buf, sem).wait()

    def start_scatter(idx_row, buf, sem):
        pltpu.async_copy(buf, acc.at[idx_row], sem, add=True)

    def wait_scatter(idx_row, buf, sem):
        pltpu.make_async_copy(buf, acc.at[idx_row], sem).wait()

    start_gather(srcv.at[0], b0, g0)
    start_gather(srcv.at[1], b1, g1)

    def seg_body(sg, _):
        b = lax.rem(sg, 2) * SEG
        nb = lax.rem(sg + 1, 2) * SEG

        @pl.when(sg + 1 < NSEG)
        def _():
            pltpu.async_copy(src_hbm.at[wid, pl.ds((sg + 1) * SEG, SEG)],
                             srcv.at[pl.ds(nb, SEG)], isem)
            pltpu.async_copy(dst_hbm.at[wid, pl.ds((sg + 1) * SEG, SEG)],
                             dstv.at[pl.ds(nb, SEG)], isem)

        for k in range(SEG):
            m = k % 4
            wait_gather(srcv.at[b + k], bufs[m], gsems[m])
            start_scatter(dstv.at[b + k], bufs[m], ssems[m])
            m2 = (k + 2) % 4
            if k >= 2:
                wait_scatter(dstv.at[b + k - 2], bufs[m2], ssems[m2])
            else:
                @pl.when(sg >= 1)
                def _():
                    wait_scatter(dstv.at[b + k], bufs[m2], ssems[m2])
            if k < SEG - 2:
                start_gather(srcv.at[b + k + 2], bufs[m2], gsems[m2])
            else:
                @pl.when(sg + 1 < NSEG)
                def _():
                    if k == SEG - 2:
                        pltpu.make_async_copy(
                            src_hbm.at[wid, pl.ds((sg + 1) * SEG, SEG)],
                            srcv.at[pl.ds(nb, SEG)], isem).wait()
                        pltpu.make_async_copy(
                            dst_hbm.at[wid, pl.ds((sg + 1) * SEG, SEG)],
                            dstv.at[pl.ds(nb, SEG)], isem).wait()
                    start_gather(srcv.at[nb + k - (SEG - 2)], bufs[m2],
                                 gsems[m2])
        return _
    lax.fori_loop(0, NSEG, seg_body, None)

    wait_scatter(dstv.at[SEG + 6], b2, s2)
    wait_scatter(dstv.at[SEG + 7], b3, s3)

    plsc.subcore_barrier()

    for k in range((N_WCHUNKS + NUM_SUBCORES - 1) // NUM_SUBCORES):
        j = s + k * NUM_SUBCORES
        buf, sem = (b0, w0) if k % 2 == 0 else (b1, w1)

        @pl.when(j < N_WCHUNKS)
        def _():
            if k >= 2:
                pltpu.make_async_copy(
                    buf.at[pl.ds(0, WCHUNK)],
                    part_hbm.at[c, pl.ds(j * WCHUNK, WCHUNK)], sem).wait()
            pltpu.sync_copy(acc.at[pl.ds(j * WCHUNK, WCHUNK)],
                            buf.at[pl.ds(0, WCHUNK)])
            pltpu.async_copy(buf.at[pl.ds(0, WCHUNK)],
                             part_hbm.at[c, pl.ds(j * WCHUNK, WCHUNK)], sem)

    for sem, buf in ((w0, b0), (w1, b1)):
        pltpu.make_async_copy(
            buf.at[pl.ds(0, WCHUNK)],
            part_hbm.at[c, pl.ds(0, WCHUNK)], sem).wait()


_sc_scatter_gather = functools.partial(
    pl.kernel,
    out_type=jax.ShapeDtypeStruct((NUM_CORES, N_NODES, D_FEAT), jnp.float32),
    mesh=plsc.VectorSubcoreMesh(core_axis_name="c", subcore_axis_name="s"),
    scratch_types=[
        pltpu.VMEM_SHARED((ACC_ROWS, D_FEAT), jnp.float32),
        pltpu.VMEM((2 * SEG, CHUNK), jnp.int32),
        pltpu.VMEM((2 * SEG, CHUNK), jnp.int32),
        pltpu.VMEM((CHUNK, D_FEAT), jnp.float32),
        pltpu.VMEM((CHUNK, D_FEAT), jnp.float32),
        pltpu.VMEM((CHUNK, D_FEAT), jnp.float32),
        pltpu.VMEM((CHUNK, D_FEAT), jnp.float32),
    ] + [pltpu.SemaphoreType.DMA] * 11,
)(_sc_kernel_body)


def _add_body(a_ref, b_ref, o_ref):
    o_ref[...] = a_ref[0] + b_ref[0]


def _combine(partials):
    rows_per_blk = N_NODES // 2
    return pl.pallas_call(
        _add_body,
        out_shape=jax.ShapeDtypeStruct((N_NODES, D_FEAT), jnp.float32),
        grid=(2,),
        in_specs=[
            pl.BlockSpec((1, rows_per_blk, D_FEAT), lambda i: (0, i, 0)),
            pl.BlockSpec((1, rows_per_blk, D_FEAT), lambda i: (1, i, 0)),
        ],
        out_specs=pl.BlockSpec((rows_per_blk, D_FEAT), lambda i: (i, 0)),
    )(partials, partials)


def kernel(x, edge_index):
    src = edge_index[0].astype(jnp.int32)
    dst = edge_index[1].astype(jnp.int32)
    pad_per_tile = PAD_EDGES // NUM_WORKERS
    real_per_tile = N_EDGES // NUM_WORKERS
    lane = jnp.arange(pad_per_tile, dtype=jnp.int32)
    pad_src = jnp.broadcast_to(lane % N_NODES, (NUM_WORKERS, pad_per_tile))
    pad_dst = jnp.broadcast_to(
        N_NODES + lane % (ACC_ROWS - N_NODES), (NUM_WORKERS, pad_per_tile))
    src = jnp.concatenate(
        [src.reshape(NUM_WORKERS, real_per_tile), pad_src], axis=1)
    dst = jnp.concatenate(
        [dst.reshape(NUM_WORKERS, real_per_tile), pad_dst], axis=1)
    src = src.reshape(NUM_WORKERS, CHUNKS_PER_TILE, CHUNK)
    dst = dst.reshape(NUM_WORKERS, CHUNKS_PER_TILE, CHUNK)
    partials = _sc_scatter_gather(src, dst, x)
    return _combine(partials)


# speedup vs baseline: 3.4110x; 1.0005x over previous
import functools

import jax
import jax.numpy as jnp
from jax import lax
from jax.experimental import pallas as pl
from jax.experimental.pallas import tpu as pltpu
from jax.experimental.pallas import tpu_sc as plsc

N_NODES = 10000
D_FEAT = 128
N_EDGES = 320000

NUM_CORES = 2
NUM_SUBCORES = 16
NUM_WORKERS = NUM_CORES * NUM_SUBCORES

CHUNK = 88
SEG = 8
NSEG = 15
CHUNKS_PER_TILE = SEG * NSEG
EDGES_PER_TILE = CHUNK * CHUNKS_PER_TILE
PAD_EDGES = NUM_WORKERS * EDGES_PER_TILE - N_EDGES

ACC_ROWS = 10120
ZCHUNK = 88
N_ZCHUNKS = ACC_ROWS // ZCHUNK
WCHUNK = 80
N_WCHUNKS = N_NODES // WCHUNK


def _sc_kernel_body(src_hbm, dst_hbm, x_hbm, part_hbm,
                    acc, srcv, dstv, b0, b1, b2, b3,
                    g0, g1, g2, g3, s0, s1, s2, s3, isem, w0, w1):
    c = lax.axis_index("c")
    s = lax.axis_index("s")
    wid = c * NUM_SUBCORES + s
    bufs = (b0, b1, b2, b3)
    gsems = (g0, g1, g2, g3)
    ssems = (s0, s1, s2, s3)
    wsems = (w0, w1)

    def zero_row(r, _):
        for k in range(D_FEAT // 16):
            b0[r, pl.ds(k * 16, 16)] = jnp.zeros((16,), jnp.float32)
        return _
    lax.fori_loop(0, ZCHUNK, zero_row, None)

    for k in range((N_ZCHUNKS + NUM_SUBCORES - 1) // NUM_SUBCORES):
        j = s + k * NUM_SUBCORES

        @pl.when(j < N_ZCHUNKS)
        def _():
            pltpu.sync_copy(b0, acc.at[pl.ds(j * ZCHUNK, ZCHUNK)])

    pltpu.sync_copy(src_hbm.at[wid, pl.ds(0, SEG)], srcv.at[pl.ds(0, SEG)])
    pltpu.sync_copy(dst_hbm.at[wid, pl.ds(0, SEG)], dstv.at[pl.ds(0, SEG)])

    plsc.subcore_barrier()

    def start_gather(idx_row, buf, sem):
        pltpu.async_copy(x_hbm.at[idx_row], buf, sem)

    def wait_gather(idx_row, buf, sem):
        pltpu.make_async_copy(x_hbm.at[idx_row], buf, sem).wait()

    def start_scatter(idx_row, buf, sem):
        pltpu.async_copy(buf, acc.at[idx_row], sem, add=True)

    def wait_scatter(idx_row, buf, sem):
        pltpu.make_async_copy(buf, acc.at[idx_row], sem).wait()

    start_gather(srcv.at[0], b0, g0)
    start_gather(srcv.at[1], b1, g1)

    def seg_body(sg, _):
        b = lax.rem(sg, 2) * SEG
        nb = lax.rem(sg + 1, 2) * SEG

        for k in range(SEG):
            m = k % 4
            wait_gather(srcv.at[b + k], bufs[m], gsems[m])
            start_scatter(dstv.at[b + k], bufs[m], ssems[m])
            m2 = (k + 2) % 4
            if k >= 2:
                wait_scatter(dstv.at[b + k - 2], bufs[m2], ssems[m2])
            else:
                @pl.when(sg >= 1)
                def _():
                    wait_scatter(dstv.at[b + k], bufs[m2], ssems[m2])
            if k == 1:
                @pl.when(sg + 1 < NSEG)
                def _():
                    pltpu.async_copy(
                        src_hbm.at[wid, pl.ds((sg + 1) * SEG, SEG)],
                        srcv.at[pl.ds(nb, SEG)], isem)
                    pltpu.async_copy(
                        dst_hbm.at[wid, pl.ds((sg + 1) * SEG, SEG)],
                        dstv.at[pl.ds(nb, SEG)], isem)
            if k < SEG - 2:
                start_gather(srcv.at[b + k + 2], bufs[m2], gsems[m2])
            else:
                @pl.when(sg + 1 < NSEG)
                def _():
                    if k == SEG - 2:
                        pltpu.make_async_copy(
                            src_hbm.at[wid, pl.ds((sg + 1) * SEG, SEG)],
                            srcv.at[pl.ds(nb, SEG)], isem).wait()
                        pltpu.make_async_copy(
                            dst_hbm.at[wid, pl.ds((sg + 1) * SEG, SEG)],
                            dstv.at[pl.ds(nb, SEG)], isem).wait()
                    start_gather(srcv.at[nb + k - (SEG - 2)], bufs[m2],
                                 gsems[m2])
        return _
    lax.fori_loop(0, NSEG, seg_body, None)

    wait_scatter(dstv.at[SEG + 6], b2, s2)
    wait_scatter(dstv.at[SEG + 7], b3, s3)

    plsc.subcore_barrier()

    for k in range((N_WCHUNKS + NUM_SUBCORES - 1) // NUM_SUBCORES):
        j = s + k * NUM_SUBCORES
        buf, sem = (b0, w0) if k % 2 == 0 else (b1, w1)

        @pl.when(j < N_WCHUNKS)
        def _():
            if k >= 2:
                pltpu.make_async_copy(
                    buf.at[pl.ds(0, WCHUNK)],
                    part_hbm.at[c, pl.ds(j * WCHUNK, WCHUNK)], sem).wait()
            pltpu.sync_copy(acc.at[pl.ds(j * WCHUNK, WCHUNK)],
                            buf.at[pl.ds(0, WCHUNK)])
            pltpu.async_copy(buf.at[pl.ds(0, WCHUNK)],
                             part_hbm.at[c, pl.ds(j * WCHUNK, WCHUNK)], sem)

    for sem, buf in ((w0, b0), (w1, b1)):
        pltpu.make_async_copy(
            buf.at[pl.ds(0, WCHUNK)],
            part_hbm.at[c, pl.ds(0, WCHUNK)], sem).wait()


_sc_scatter_gather = functools.partial(
    pl.kernel,
    out_type=jax.ShapeDtypeStruct((NUM_CORES, N_NODES, D_FEAT), jnp.float32),
    mesh=plsc.VectorSubcoreMesh(core_axis_name="c", subcore_axis_name="s"),
    scratch_types=[
        pltpu.VMEM_SHARED((ACC_ROWS, D_FEAT), jnp.float32),
        pltpu.VMEM((2 * SEG, CHUNK), jnp.int32),
        pltpu.VMEM((2 * SEG, CHUNK), jnp.int32),
        pltpu.VMEM((CHUNK, D_FEAT), jnp.float32),
        pltpu.VMEM((CHUNK, D_FEAT), jnp.float32),
        pltpu.VMEM((CHUNK, D_FEAT), jnp.float32),
        pltpu.VMEM((CHUNK, D_FEAT), jnp.float32),
    ] + [pltpu.SemaphoreType.DMA] * 11,
)(_sc_kernel_body)


def _add_body(a_ref, b_ref, o_ref):
    o_ref[...] = a_ref[0] + b_ref[0]


def _combine(partials):
    rows_per_blk = N_NODES // 2
    return pl.pallas_call(
        _add_body,
        out_shape=jax.ShapeDtypeStruct((N_NODES, D_FEAT), jnp.float32),
        grid=(2,),
        in_specs=[
            pl.BlockSpec((1, rows_per_blk, D_FEAT), lambda i: (0, i, 0)),
            pl.BlockSpec((1, rows_per_blk, D_FEAT), lambda i: (1, i, 0)),
        ],
        out_specs=pl.BlockSpec((rows_per_blk, D_FEAT), lambda i: (i, 0)),
    )(partials, partials)


def kernel(x, edge_index):
    src = edge_index[0].astype(jnp.int32)
    dst = edge_index[1].astype(jnp.int32)
    pad_per_tile = PAD_EDGES // NUM_WORKERS
    real_per_tile = N_EDGES // NUM_WORKERS
    lane = jnp.arange(pad_per_tile, dtype=jnp.int32)
    pad_src = jnp.broadcast_to(lane % N_NODES, (NUM_WORKERS, pad_per_tile))
    pad_dst = jnp.broadcast_to(
        N_NODES + lane % (ACC_ROWS - N_NODES), (NUM_WORKERS, pad_per_tile))
    src = jnp.concatenate(
        [src.reshape(NUM_WORKERS, real_per_tile), pad_src], axis=1)
    dst = jnp.concatenate(
        [dst.reshape(NUM_WORKERS, real_per_tile), pad_dst], axis=1)
    src = src.reshape(NUM_WORKERS, CHUNKS_PER_TILE, CHUNK)
    dst = dst.reshape(NUM_WORKERS, CHUNKS_PER_TILE, CHUNK)
    partials = _sc_scatter_gather(src, dst, x)
    return _combine(partials)
